# baseline (speedup 1.0000x reference)
import functools

import jax
import jax.numpy as jnp
from jax import lax
from jax.experimental import pallas as pl
from jax.experimental.pallas import tpu as pltpu
from jax.experimental.pallas import tpu_sc as plsc

_N_E = 512
_E_DIM = 32
_BETA = 0.25
_N_TOK = 131072


_BZ = 2048
_NB = _N_TOK // _BZ


def _rowsum32(t):
    u = ((t[:, 0:8] + t[:, 8:16]) + t[:, 16:24]) + t[:, 24:32]
    v = u[:, 0:4] + u[:, 4:8]
    w = v[:, 0:2] + v[:, 2:4]
    return w[:, 0:1] + w[:, 1:2]


def _tc_body(z_ref, emb_ref, idx_ref, acc_ref, en_s):
    z = z_ref[...]
    mm = lax.dot_general(z, emb_ref[...], (((1,), (1,)), ((), ())),
                         preferred_element_type=jnp.float32,
                         precision="default")
    zn = _rowsum32(z * z)

    @pl.when(pl.program_id(0) == 0)
    def _():
        emb = emb_ref[...]
        en_s[...] = _rowsum32(emb * emb).reshape(1, _N_E)

    en = en_s[...]
    col0 = lax.broadcasted_iota(jnp.int32, (_BZ, 128), 1)
    bv = (zn + en[:, 0:128]) - 2.0 * mm[:, 0:128]
    bi = col0
    for c in range(1, _N_E // 128):
        lo = 128 * c
        nv = (zn + en[:, lo:lo + 128]) - 2.0 * mm[:, lo:lo + 128]
        take = nv < bv
        bv = jnp.where(take, nv, bv)
        bi = jnp.where(take, col0 + lo, bi)
    for sh in (64, 32, 16, 8, 4, 2, 1):
        av, ai = bv[:, 0:sh], bi[:, 0:sh]
        cv, ci = bv[:, sh:2 * sh], bi[:, sh:2 * sh]
        take = (cv < av) | ((cv == av) & (ci < ai))
        bv = jnp.where(take, cv, av)
        bi = jnp.where(take, ci, ai)
    idx_ref[0, 0, :] = bi[:, 0]
    acc_ref[0] = jnp.broadcast_to(jnp.sum(bv[:, 0:1]).reshape(1, 1),
                                  (8, 128))


_tc_call = pl.pallas_call(
    _tc_body,
    grid=(_NB,),
    in_specs=[
        pl.BlockSpec((_BZ, _E_DIM), lambda i: (i, 0)),
        pl.BlockSpec((_N_E, _E_DIM), lambda i: (0, 0)),
    ],
    out_specs=[
        pl.BlockSpec((1, 1, _BZ), lambda i: (i, 0, 0)),
        pl.BlockSpec((1, 8, 128), lambda i: (i, 0, 0)),
    ],
    out_shape=[
        jax.ShapeDtypeStruct((_NB, 1, _BZ), jnp.int32),
        jax.ShapeDtypeStruct((_NB, 8, 128), jnp.float32),
    ],
    scratch_shapes=[
        pltpu.VMEM((1, _N_E), jnp.float32),
    ],
    compiler_params=pltpu.CompilerParams(
        dimension_semantics=("arbitrary",)),
)


_NC = 2
_NS = 16
_NW = _NC * _NS
_BPW = _N_TOK // _NW
_CH = 1024
_NCH = _BPW // _CH


def _sc_gather_body(emb_hbm, idx_hbm, out_hbm, idx_v, buf_a, buf_b, sem_a,
                    sem_b, osem):
    wid = lax.axis_index("s") * _NC + lax.axis_index("c")
    base = wid * _BPW
    bufs = (buf_a, buf_b)
    sems = (sem_a, sem_b)
    pltpu.sync_copy(idx_hbm.at[pl.ds(base, _BPW)], idx_v)
    gathers = [pltpu.async_copy(emb_hbm.at[idx_v.at[pl.ds(0, _CH)]],
                                bufs[0], sems[0])]
    writes = [None, None]
    for c in range(_NCH):
        b = c % 2
        nb = (c + 1) % 2
        if c + 1 < _NCH:
            off = (c + 1) * _CH
            if writes[nb] is not None:
                writes[nb].wait()
                writes[nb] = None
            gathers.append(
                pltpu.async_copy(emb_hbm.at[idx_v.at[pl.ds(off, _CH)]],
                                 bufs[nb], sems[nb]))
        gathers[c].wait()
        writes[b] = pltpu.async_copy(bufs[b],
                                     out_hbm.at[pl.ds(base + c * _CH, _CH)],
                                     osem)
    for w in writes:
        if w is not None:
            w.wait()


_sc_gather = pl.kernel(
    _sc_gather_body,
    mesh=plsc.VectorSubcoreMesh(core_axis_name="c", subcore_axis_name="s"),
    out_type=jax.ShapeDtypeStruct((_N_TOK, _E_DIM), jnp.float32),
    compiler_params=pltpu.CompilerParams(use_tc_tiling_on_sc=False),
    scratch_types=[
        pltpu.VMEM((_BPW,), jnp.int32),
        pltpu.VMEM((_CH, _E_DIM), jnp.float32),
        pltpu.VMEM((_CH, _E_DIM), jnp.float32),
        pltpu.SemaphoreType.DMA,
        pltpu.SemaphoreType.DMA,
        pltpu.SemaphoreType.DMA,
    ],
)


def kernel(z, emb):
    idx3, acc = _tc_call(z, emb)
    idx = idx3.reshape(_N_TOK)
    z_q = _sc_gather(emb, idx)
    loss = jnp.sum(acc[:, 0, 0]) * ((1.0 + _BETA) / (_N_TOK * _E_DIM))
    return (z_q, loss)


# speedup vs baseline: 1.4995x; 1.4995x over previous
import functools

import jax
import jax.numpy as jnp
from jax import lax
from jax.experimental import pallas as pl
from jax.experimental.pallas import tpu as pltpu
from jax.experimental.pallas import tpu_sc as plsc

_N_E = 512
_E_DIM = 32
_BETA = 0.25
_N_TOK = 131072


_BZ = 2048
_NB = _N_TOK // _BZ


def _rowsum32(t):
    u = ((t[:, 0:8] + t[:, 8:16]) + t[:, 16:24]) + t[:, 24:32]
    v = u[:, 0:4] + u[:, 4:8]
    w = v[:, 0:2] + v[:, 2:4]
    return w[:, 0:1] + w[:, 1:2]


def _tc_body(z_ref, emb_ref, idx_ref, acc_ref, en_s):
    z = z_ref[...]
    mm = lax.dot_general(z, emb_ref[...], (((1,), (1,)), ((), ())),
                         preferred_element_type=jnp.float32,
                         precision="default")
    zn = _rowsum32(z * z)

    @pl.when(pl.program_id(0) == 0)
    def _():
        emb = emb_ref[...]
        en_s[...] = _rowsum32(emb * emb).reshape(1, _N_E)

    en = en_s[...]
    d = (zn + en) - 2.0 * mm
    m = jnp.min(d, axis=1, keepdims=True)
    col = lax.broadcasted_iota(jnp.int32, (_BZ, _N_E), 1)
    idx = jnp.min(jnp.where(d == m, col, _N_E), axis=1)
    idx_ref[0, 0, :] = idx.astype(jnp.int32)
    acc_ref[0] = jnp.broadcast_to(jnp.sum(m).reshape(1, 1), (8, 128))


_tc_call = pl.pallas_call(
    _tc_body,
    grid=(_NB,),
    in_specs=[
        pl.BlockSpec((_BZ, _E_DIM), lambda i: (i, 0)),
        pl.BlockSpec((_N_E, _E_DIM), lambda i: (0, 0)),
    ],
    out_specs=[
        pl.BlockSpec((1, 1, _BZ), lambda i: (i, 0, 0)),
        pl.BlockSpec((1, 8, 128), lambda i: (i, 0, 0)),
    ],
    out_shape=[
        jax.ShapeDtypeStruct((_NB, 1, _BZ), jnp.int32),
        jax.ShapeDtypeStruct((_NB, 8, 128), jnp.float32),
    ],
    scratch_shapes=[
        pltpu.VMEM((1, _N_E), jnp.float32),
    ],
    compiler_params=pltpu.CompilerParams(
        dimension_semantics=("arbitrary",)),
)


_NC = 2
_NS = 16
_NW = _NC * _NS
_BPW = _N_TOK // _NW
_CH = 1024
_NCH = _BPW // _CH


def _sc_gather_body(emb_hbm, idx_hbm, out_hbm, idx_v, buf_a, buf_b, sem_a,
                    sem_b, osem):
    wid = lax.axis_index("s") * _NC + lax.axis_index("c")
    base = wid * _BPW
    bufs = (buf_a, buf_b)
    sems = (sem_a, sem_b)
    pltpu.sync_copy(idx_hbm.at[pl.ds(base, _BPW)], idx_v)
    gathers = [pltpu.async_copy(emb_hbm.at[idx_v.at[pl.ds(0, _CH)]],
                                bufs[0], sems[0])]
    writes = [None, None]
    for c in range(_NCH):
        b = c % 2
        nb = (c + 1) % 2
        if c + 1 < _NCH:
            off = (c + 1) * _CH
            if writes[nb] is not None:
                writes[nb].wait()
                writes[nb] = None
            gathers.append(
                pltpu.async_copy(emb_hbm.at[idx_v.at[pl.ds(off, _CH)]],
                                 bufs[nb], sems[nb]))
        gathers[c].wait()
        writes[b] = pltpu.async_copy(bufs[b],
                                     out_hbm.at[pl.ds(base + c * _CH, _CH)],
                                     osem)
    for w in writes:
        if w is not None:
            w.wait()


_sc_gather = pl.kernel(
    _sc_gather_body,
    mesh=plsc.VectorSubcoreMesh(core_axis_name="c", subcore_axis_name="s"),
    out_type=jax.ShapeDtypeStruct((_N_TOK, _E_DIM), jnp.float32),
    compiler_params=pltpu.CompilerParams(use_tc_tiling_on_sc=False),
    scratch_types=[
        pltpu.VMEM((_BPW,), jnp.int32),
        pltpu.VMEM((_CH, _E_DIM), jnp.float32),
        pltpu.VMEM((_CH, _E_DIM), jnp.float32),
        pltpu.SemaphoreType.DMA,
        pltpu.SemaphoreType.DMA,
        pltpu.SemaphoreType.DMA,
    ],
)


def kernel(z, emb):
    idx3, acc = _tc_call(z, emb)
    idx = idx3.reshape(_N_TOK)
    z_q = _sc_gather(emb, idx)
    loss = jnp.sum(acc[:, 0, 0]) * ((1.0 + _BETA) / (_N_TOK * _E_DIM))
    return (z_q, loss)


# speedup vs baseline: 1.5217x; 1.0148x over previous
import functools

import jax
import jax.numpy as jnp
from jax import lax
from jax.experimental import pallas as pl
from jax.experimental.pallas import tpu as pltpu
from jax.experimental.pallas import tpu_sc as plsc

_N_E = 512
_E_DIM = 32
_BETA = 0.25
_N_TOK = 131072


_BZ = 4096
_NB = _N_TOK // _BZ


def _rowsum32(t):
    u = ((t[:, 0:8] + t[:, 8:16]) + t[:, 16:24]) + t[:, 24:32]
    v = u[:, 0:4] + u[:, 4:8]
    w = v[:, 0:2] + v[:, 2:4]
    return w[:, 0:1] + w[:, 1:2]


def _tc_body(z_ref, emb_ref, idx_ref, acc_ref, en_s):
    z = z_ref[...]
    mm = lax.dot_general(z, emb_ref[...], (((1,), (1,)), ((), ())),
                         preferred_element_type=jnp.float32,
                         precision="default")
    zn = _rowsum32(z * z)

    @pl.when(pl.program_id(0) == 0)
    def _():
        emb = emb_ref[...]
        en_s[...] = _rowsum32(emb * emb).reshape(1, _N_E)

    en = en_s[...]
    d = (zn + en) - 2.0 * mm
    m = jnp.min(d, axis=1, keepdims=True)
    col = lax.broadcasted_iota(jnp.int32, (_BZ, _N_E), 1)
    idx = jnp.min(jnp.where(d == m, col, _N_E), axis=1)
    idx_ref[0, 0, :] = idx.astype(jnp.int32)
    acc_ref[0] = jnp.broadcast_to(jnp.sum(m).reshape(1, 1), (8, 128))


_tc_call = pl.pallas_call(
    _tc_body,
    grid=(_NB,),
    in_specs=[
        pl.BlockSpec((_BZ, _E_DIM), lambda i: (i, 0)),
        pl.BlockSpec((_N_E, _E_DIM), lambda i: (0, 0)),
    ],
    out_specs=[
        pl.BlockSpec((1, 1, _BZ), lambda i: (i, 0, 0)),
        pl.BlockSpec((1, 8, 128), lambda i: (i, 0, 0)),
    ],
    out_shape=[
        jax.ShapeDtypeStruct((_NB, 1, _BZ), jnp.int32),
        jax.ShapeDtypeStruct((_NB, 8, 128), jnp.float32),
    ],
    scratch_shapes=[
        pltpu.VMEM((1, _N_E), jnp.float32),
    ],
    compiler_params=pltpu.CompilerParams(
        dimension_semantics=("arbitrary",)),
)


_NC = 2
_NS = 16
_NW = _NC * _NS
_BPW = _N_TOK // _NW
_CH = 1024
_NCH = _BPW // _CH


def _sc_gather_body(emb_hbm, idx_hbm, out_hbm, idx_v, buf_a, buf_b, sem_a,
                    sem_b, osem):
    wid = lax.axis_index("s") * _NC + lax.axis_index("c")
    base = wid * _BPW
    bufs = (buf_a, buf_b)
    sems = (sem_a, sem_b)
    pltpu.sync_copy(idx_hbm.at[pl.ds(base, _BPW)], idx_v)
    gathers = [pltpu.async_copy(emb_hbm.at[idx_v.at[pl.ds(0, _CH)]],
                                bufs[0], sems[0])]
    writes = [None, None]
    for c in range(_NCH):
        b = c % 2
        nb = (c + 1) % 2
        if c + 1 < _NCH:
            off = (c + 1) * _CH
            if writes[nb] is not None:
                writes[nb].wait()
                writes[nb] = None
            gathers.append(
                pltpu.async_copy(emb_hbm.at[idx_v.at[pl.ds(off, _CH)]],
                                 bufs[nb], sems[nb]))
        gathers[c].wait()
        writes[b] = pltpu.async_copy(bufs[b],
                                     out_hbm.at[pl.ds(base + c * _CH, _CH)],
                                     osem)
    for w in writes:
        if w is not None:
            w.wait()


_sc_gather = pl.kernel(
    _sc_gather_body,
    mesh=plsc.VectorSubcoreMesh(core_axis_name="c", subcore_axis_name="s"),
    out_type=jax.ShapeDtypeStruct((_N_TOK, _E_DIM), jnp.float32),
    compiler_params=pltpu.CompilerParams(use_tc_tiling_on_sc=False),
    scratch_types=[
        pltpu.VMEM((_BPW,), jnp.int32),
        pltpu.VMEM((_CH, _E_DIM), jnp.float32),
        pltpu.VMEM((_CH, _E_DIM), jnp.float32),
        pltpu.SemaphoreType.DMA,
        pltpu.SemaphoreType.DMA,
        pltpu.SemaphoreType.DMA,
    ],
)


def kernel(z, emb):
    idx3, acc = _tc_call(z, emb)
    idx = idx3.reshape(_N_TOK)
    z_q = _sc_gather(emb, idx)
    loss = jnp.sum(acc[:, 0, 0]) * ((1.0 + _BETA) / (_N_TOK * _E_DIM))
    return (z_q, loss)
